# baseline (speedup 1.0000x reference)
import functools

import jax
import jax.numpy as jnp
import numpy as np
from jax import lax
from jax.experimental import pallas as pl
from jax.experimental.pallas import tpu as pltpu
from jax.experimental.pallas import tpu_sc as plsc

N = 16384
EMB = 128
H2 = 512
OUT = 256
NW = 32
RPW = N // NW
IDX_W = 128
CHUNKS = RPW // IDX_W
SCL = 16
BLK = 2048

_sc_mesh = plsc.VectorSubcoreMesh(core_axis_name="c", subcore_axis_name="s")


@functools.partial(
    pl.kernel,
    mesh=_sc_mesh,
    compiler_params=pltpu.CompilerParams(needs_layout_passes=False),
    out_type=(
        jax.ShapeDtypeStruct((N, EMB), jnp.bfloat16),
        jax.ShapeDtypeStruct((N, EMB), jnp.bfloat16),
    ),
    scratch_types=[
        pltpu.VMEM((CHUNKS, IDX_W), jnp.int32),
        pltpu.VMEM((2, IDX_W, EMB), jnp.float32),
        pltpu.VMEM((RPW, EMB), jnp.bfloat16),
        pltpu.SemaphoreType.DMA,
        pltpu.SemaphoreType.DMA,
    ],
)
def _sc_gather(gtab, ktab, gidx, kidx, gout, kout, idx_v, rows_v, out16_v,
               sem0, sem1):
    wid = lax.axis_index("s") * 2 + lax.axis_index("c")
    row0 = wid * RPW
    blk0 = wid * CHUNKS
    sems = (sem0, sem1)

    def convert_chunk(j):
        buf = j % 2

        def row_body(r, carry):
            for cgrp in range(EMB // (2 * SCL)):
                a = rows_v[buf, r, pl.ds(cgrp * 2 * SCL, SCL)]
                b = rows_v[buf, r, pl.ds(cgrp * 2 * SCL + SCL, SCL)]
                p = plsc.pack(a, b, format=plsc.PackFormat.INTERLEAVED)
                out16_v[j * IDX_W + r, pl.ds(cgrp * 2 * SCL, 2 * SCL)] = p
            return carry

        lax.fori_loop(0, IDX_W, row_body, 0)

    def one_table(tab, out_hbm, idx_hbm):
        pltpu.sync_copy(idx_hbm.at[pl.ds(blk0, CHUNKS)], idx_v)
        copies = [None] * CHUNKS
        copies[0] = pltpu.async_copy(
            tab.at[idx_v.at[0]], rows_v.at[0], sems[0]
        )
        for j in range(CHUNKS):
            if j + 1 < CHUNKS:
                copies[j + 1] = pltpu.async_copy(
                    tab.at[idx_v.at[j + 1]], rows_v.at[(j + 1) % 2],
                    sems[(j + 1) % 2],
                )
            copies[j].wait()
            convert_chunk(j)
        pltpu.sync_copy(out16_v, out_hbm.at[pl.ds(row0, RPW)])

    one_table(gtab, gout, gidx)
    one_table(ktab, kout, kidx)


def _pack_perm() -> np.ndarray:
    perm = np.empty(EMB, dtype=np.int32)
    for grp in range(EMB // (2 * SCL)):
        for i in range(SCL):
            perm[32 * grp + 2 * i] = 32 * grp + i
            perm[32 * grp + 2 * i + 1] = 32 * grp + SCL + i
    return perm


_W1_PERM = np.concatenate([_pack_perm(), EMB + _pack_perm()])


def _mlp_body(g_ref, k_ref, w1_ref, w2_ref, b1_ref, b2_ref, o_ref):
    dnums = (((1,), (1,)), ((), ()))
    bf = jnp.bfloat16
    emb = jnp.concatenate([g_ref[...], k_ref[...]], axis=1)
    h = lax.dot_general(
        emb, w1_ref[...].astype(bf), dnums, preferred_element_type=jnp.float32
    )
    h16 = jnp.maximum(h.astype(bf) + b1_ref[...].astype(bf), jnp.asarray(0, bf))
    o_ref[...] = (
        lax.dot_general(
            h16, w2_ref[...].astype(bf), dnums, preferred_element_type=jnp.float32
        )
        + b2_ref[...]
    )


def _mlp(gbuf, kbuf, w1, w2, b1, b2):
    return pl.pallas_call(
        _mlp_body,
        grid=(N // BLK,),
        in_specs=[
            pl.BlockSpec((BLK, EMB), lambda i: (i, 0)),
            pl.BlockSpec((BLK, EMB), lambda i: (i, 0)),
            pl.BlockSpec((H2, 2 * EMB), lambda i: (0, 0)),
            pl.BlockSpec((OUT, H2), lambda i: (0, 0)),
            pl.BlockSpec((1, H2), lambda i: (0, 0)),
            pl.BlockSpec((1, OUT), lambda i: (0, 0)),
        ],
        out_specs=pl.BlockSpec((BLK, OUT), lambda i: (i, 0)),
        out_shape=jax.ShapeDtypeStruct((N, OUT), jnp.float32),
    )(gbuf, kbuf, w1, w2, b1, b2)


def kernel(input_tensor, genre_table, key_table, W1, b1, W2, b2):
    g_idx = input_tensor[:, 0].reshape(N // IDX_W, IDX_W)
    k_idx = input_tensor[:, 1].reshape(N // IDX_W, IDX_W)
    gbuf, kbuf = _sc_gather(genre_table, key_table, g_idx, k_idx)
    w1p = W1[:, jnp.asarray(_W1_PERM)]
    return _mlp(gbuf, kbuf, w1p, W2, b1.reshape(1, H2), b2.reshape(1, OUT))


# speedup vs baseline: 1.2616x; 1.2616x over previous
import functools

import jax
import jax.numpy as jnp
import numpy as np
from jax import lax
from jax.experimental import pallas as pl
from jax.experimental.pallas import tpu as pltpu
from jax.experimental.pallas import tpu_sc as plsc

N = 16384
EMB = 128
H2 = 512
OUT = 256
NW = 32
RPW = N // NW
IDX_W = 128
CHUNKS = RPW // IDX_W
SCL = 16
BLK = 2048

_sc_mesh = plsc.VectorSubcoreMesh(core_axis_name="c", subcore_axis_name="s")


VOCAB = 1000


@functools.partial(
    pl.kernel,
    mesh=_sc_mesh,
    out_type=(
        jax.ShapeDtypeStruct((N, EMB), jnp.float32),
        jax.ShapeDtypeStruct((N, EMB), jnp.float32),
    ),
    scratch_types=[
        pltpu.VMEM((CHUNKS, IDX_W), jnp.int32),
        pltpu.VMEM((RPW, EMB), jnp.float32),
        pltpu.VMEM_SHARED((VOCAB, EMB), jnp.float32),
        pltpu.VMEM_SHARED((VOCAB, EMB), jnp.float32),
        pltpu.SemaphoreType.DMA,
    ],
)
def _sc_gather(gtab, ktab, gidx, kidx, gout, kout, idx_v, rows_v, spm_g,
               spm_k, sem):
    wid = lax.axis_index("s") * 2 + lax.axis_index("c")
    row0 = wid * RPW
    blk0 = wid * CHUNKS

    @pl.when(lax.axis_index("s") == 0)
    def _stage():
        pltpu.sync_copy(gtab, spm_g)
        pltpu.sync_copy(ktab, spm_k)

    plsc.subcore_barrier()

    def one_table(tab, out_hbm, idx_hbm):
        pltpu.sync_copy(idx_hbm.at[pl.ds(blk0, CHUNKS)], idx_v)
        copies = []
        for j in range(CHUNKS):
            copies.append(
                pltpu.async_copy(
                    tab.at[idx_v.at[j]], rows_v.at[pl.ds(j * IDX_W, IDX_W)], sem
                )
            )
        for c in copies:
            c.wait()
        pltpu.sync_copy(rows_v, out_hbm.at[pl.ds(row0, RPW)])

    one_table(spm_g, gout, gidx)
    one_table(spm_k, kout, kidx)


def _mlp_body(g_ref, k_ref, w1_ref, w2_ref, b1_ref, b2_ref, o_ref):
    dnums = (((1,), (1,)), ((), ()))
    bf = jnp.bfloat16
    emb = jnp.concatenate(
        [g_ref[...].astype(bf), k_ref[...].astype(bf)], axis=1
    )
    h = lax.dot_general(
        emb, w1_ref[...].astype(bf), dnums, preferred_element_type=jnp.float32
    )
    h16 = jnp.maximum(h.astype(bf) + b1_ref[...].astype(bf), jnp.asarray(0, bf))
    o_ref[...] = (
        lax.dot_general(
            h16, w2_ref[...].astype(bf), dnums, preferred_element_type=jnp.float32
        )
        + b2_ref[...]
    )


def _mlp(gbuf, kbuf, w1, w2, b1, b2):
    return pl.pallas_call(
        _mlp_body,
        grid=(N // BLK,),
        in_specs=[
            pl.BlockSpec((BLK, EMB), lambda i: (i, 0)),
            pl.BlockSpec((BLK, EMB), lambda i: (i, 0)),
            pl.BlockSpec((H2, 2 * EMB), lambda i: (0, 0)),
            pl.BlockSpec((OUT, H2), lambda i: (0, 0)),
            pl.BlockSpec((1, H2), lambda i: (0, 0)),
            pl.BlockSpec((1, OUT), lambda i: (0, 0)),
        ],
        out_specs=pl.BlockSpec((BLK, OUT), lambda i: (i, 0)),
        out_shape=jax.ShapeDtypeStruct((N, OUT), jnp.float32),
    )(gbuf, kbuf, w1, w2, b1, b2)


def kernel(input_tensor, genre_table, key_table, W1, b1, W2, b2):
    g_idx = input_tensor[:, 0].reshape(N // IDX_W, IDX_W)
    k_idx = input_tensor[:, 1].reshape(N // IDX_W, IDX_W)
    gbuf, kbuf = _sc_gather(genre_table, key_table, g_idx, k_idx)
    return _mlp(gbuf, kbuf, W1, W2, b1.reshape(1, H2), b2.reshape(1, OUT))
